# baseline (speedup 1.0000x reference)
import functools

import jax
import jax.numpy as jnp
from jax import lax
from jax.experimental import pallas as pl
from jax.experimental.pallas import tpu as pltpu
from jax.experimental.pallas import tpu_sc as plsc

BATCH = 4096
HIST = 50
D = 128
NW = 32
PER_W = BATCH // NW
CB = 8
NST = PER_W // CB

_mesh = plsc.VectorSubcoreMesh(core_axis_name="c", subcore_axis_name="s")


@functools.partial(
    pl.kernel,
    mesh=_mesh,
    out_type=jax.ShapeDtypeStruct((HIST, BATCH, D), jnp.float32),
    scratch_types=[
        pltpu.VMEM((PER_W, HIST), jnp.int32),
        pltpu.VMEM((CB, HIST, D), jnp.float32),
        pltpu.VMEM((CB, HIST, D), jnp.float32),
        pltpu.SemaphoreType.DMA,
        pltpu.SemaphoreType.DMA,
        pltpu.SemaphoreType.DMA,
        pltpu.SemaphoreType.DMA,
    ],
)
def _gather_kernel(idx_hbm, table_hbm, out_hbm, idx_v, rows0, rows1,
                   gs0, gs1, ss0, ss1):
    wid = lax.axis_index("s") * 2 + lax.axis_index("c")
    base = wid * PER_W
    pltpu.sync_copy(idx_hbm.at[pl.ds(base, PER_W)], idx_v)

    class gather:
        def __init__(self, j, buf, sem):
            self.copies = [
                pltpu.make_async_copy(
                    table_hbm.at[idx_v.at[j * CB + b]], buf.at[b], sem)
                for b in range(CB)
            ]

        def start(self):
            for c in self.copies:
                c.start()

        def wait(self):
            for c in self.copies:
                c.wait()

    class scatter:
        def __init__(self, j, buf, sem):
            self.copies = [
                pltpu.make_async_copy(
                    buf.at[b], out_hbm.at[:, base + j * CB + b, :], sem)
                for b in range(CB)
            ]

        def start(self):
            for c in self.copies:
                c.start()

        def wait(self):
            for c in self.copies:
                c.wait()

    gather(0, rows0, gs0).start()
    gather(0, rows0, gs0).wait()
    scatter(0, rows0, ss0).start()
    gather(1, rows1, gs1).start()

    def body(g, carry):
        j = 2 * g + 1
        gather(j, rows1, gs1).wait()
        scatter(j, rows1, ss1).start()
        scatter(j - 1, rows0, ss0).wait()
        gather(j + 1, rows0, gs0).start()

        gather(j + 1, rows0, gs0).wait()
        scatter(j + 1, rows0, ss0).start()
        scatter(j, rows1, ss1).wait()
        gather(j + 2, rows1, gs1).start()
        return carry

    lax.fori_loop(0, NST // 2 - 1, body, 0)

    j_last = NST - 1
    gather(j_last, rows1, gs1).wait()
    scatter(j_last, rows1, ss1).start()
    scatter(j_last - 1, rows0, ss0).wait()
    scatter(j_last, rows1, ss1).wait()


def kernel(x, table):
    out_t = _gather_kernel(x.astype(jnp.int32), table)
    return out_t.transpose(1, 0, 2)


# speedup vs baseline: 1.0231x; 1.0231x over previous
import functools

import jax
import jax.numpy as jnp
from jax import lax
from jax.experimental import pallas as pl
from jax.experimental.pallas import tpu as pltpu
from jax.experimental.pallas import tpu_sc as plsc

BATCH = 4096
HIST = 50
D = 128
NW = 32
PER_W = BATCH // NW
CB = 8
NST = PER_W // CB

_mesh = plsc.VectorSubcoreMesh(core_axis_name="c", subcore_axis_name="s")


@functools.partial(
    pl.kernel,
    mesh=_mesh,
    out_type=jax.ShapeDtypeStruct((HIST, BATCH, D), jnp.float32),
    scratch_types=[
        pltpu.VMEM((PER_W, HIST), jnp.int32),
        pltpu.VMEM((CB, HIST, D), jnp.float32),
        pltpu.VMEM((CB, HIST, D), jnp.float32),
        pltpu.SemaphoreType.DMA,
        pltpu.SemaphoreType.DMA,
        pltpu.SemaphoreType.DMA,
        pltpu.SemaphoreType.DMA,
    ],
)
def _gather_kernel(idx_hbm, table_hbm, out_hbm, idx_v, rows0, rows1,
                   gs0, gs1, ss0, ss1):
    wid = lax.axis_index("s") * 2 + lax.axis_index("c")
    base = wid * PER_W
    pltpu.sync_copy(idx_hbm.at[pl.ds(base, PER_W)], idx_v)

    class gather:
        def __init__(self, j, buf, sem):
            self.copies = [
                pltpu.make_async_copy(
                    table_hbm.at[idx_v.at[j * CB + b]], buf.at[b], sem)
                for b in range(CB)
            ]

        def start(self):
            for c in self.copies:
                c.start()

        def wait(self):
            for c in self.copies:
                c.wait()

    class scatter:
        def __init__(self, j, buf, sem):
            self.copies = [
                pltpu.make_async_copy(
                    buf.at[b], out_hbm.at[:, base + j * CB + b, :], sem)
                for b in range(CB)
            ]

        def start(self):
            for c in self.copies:
                c.start()

        def wait(self):
            for c in self.copies:
                c.wait()

    gather(0, rows0, gs0).start()
    gather(1, rows1, gs1).start()
    gather(0, rows0, gs0).wait()
    scatter(0, rows0, ss0).start()

    def body(g, carry):
        j = 2 * g + 1
        scatter(j - 1, rows0, ss0).wait()
        gather(j + 1, rows0, gs0).start()
        gather(j, rows1, gs1).wait()
        scatter(j, rows1, ss1).start()

        scatter(j, rows1, ss1).wait()
        gather(j + 2, rows1, gs1).start()
        gather(j + 1, rows0, gs0).wait()
        scatter(j + 1, rows0, ss0).start()
        return carry

    lax.fori_loop(0, NST // 2 - 1, body, 0)

    j_last = NST - 1
    gather(j_last, rows1, gs1).wait()
    scatter(j_last, rows1, ss1).start()
    scatter(j_last - 1, rows0, ss0).wait()
    scatter(j_last, rows1, ss1).wait()


def kernel(x, table):
    out_t = _gather_kernel(x.astype(jnp.int32), table)
    return out_t.transpose(1, 0, 2)
